# baseline (speedup 1.0000x reference)
import jax
import jax.numpy as jnp
from jax.experimental import pallas as pl
from jax.experimental.pallas import tpu as pltpu

_N = 10000
_COL = 1
_NC = 25
_CROWS = _N // _NC


def _col_sum_manual(x_hbm, o_ref, buf, sems):
    copies = []
    for k in range(_NC):
        c = pltpu.make_async_copy(
            x_hbm.at[pl.ds(k * _CROWS, _CROWS), pl.ds(0, 128)],
            buf.at[k],
            sems.at[k],
        )
        c.start()
        copies.append(c)
    acc = jnp.zeros((1, 128), jnp.float32)
    for k in range(_NC):
        copies[k].wait()
        acc = acc + jnp.sum(buf[k], axis=0, keepdims=True)
    o_ref[0, 0] = jnp.sqrt(acc[0, _COL])


def kernel(node_attr, edgeij_pair, edge_attr, g, batch):
    out = pl.pallas_call(
        _col_sum_manual,
        in_specs=[pl.BlockSpec(memory_space=pl.ANY)],
        out_specs=pl.BlockSpec(memory_space=pltpu.SMEM),
        out_shape=jax.ShapeDtypeStruct((1, 1), jnp.float32),
        scratch_shapes=[
            pltpu.VMEM((_NC, _CROWS, 128), jnp.float32),
            pltpu.SemaphoreType.DMA((_NC,)),
        ],
    )(node_attr)
    return out[0, 0]


# speedup vs baseline: 1.0365x; 1.0365x over previous
import jax
import jax.numpy as jnp
from jax.experimental import pallas as pl
from jax.experimental.pallas import tpu as pltpu

_N = 10000
_COL = 1
_NC = 10
_CROWS = _N // _NC


def _col_sum_manual(x_hbm, o_ref, buf, sems):
    copies = []
    for k in range(_NC):
        c = pltpu.make_async_copy(
            x_hbm.at[pl.ds(k * _CROWS, _CROWS), pl.ds(0, 128)],
            buf.at[k],
            sems.at[k],
        )
        c.start()
        copies.append(c)
    acc = jnp.zeros((1, 128), jnp.float32)
    for k in range(_NC):
        copies[k].wait()
        acc = acc + jnp.sum(buf[k], axis=0, keepdims=True)
    o_ref[0, 0] = jnp.sqrt(acc[0, _COL])


def kernel(node_attr, edgeij_pair, edge_attr, g, batch):
    out = pl.pallas_call(
        _col_sum_manual,
        in_specs=[pl.BlockSpec(memory_space=pl.ANY)],
        out_specs=pl.BlockSpec(memory_space=pltpu.SMEM),
        out_shape=jax.ShapeDtypeStruct((1, 1), jnp.float32),
        scratch_shapes=[
            pltpu.VMEM((_NC, _CROWS, 128), jnp.float32),
            pltpu.SemaphoreType.DMA((_NC,)),
        ],
    )(node_attr)
    return out[0, 0]
